# Initial kernel scaffold; baseline (speedup 1.0000x reference)
import jax
import jax.numpy as jnp
from jax.experimental import pallas as pl


def kernel(score_neg, score_pos):
    raise NotImplementedError("write your pallas kernel here")


# speedup vs baseline: 378.3661x; 378.3661x over previous
import functools

import jax
import jax.numpy as jnp
from jax import lax
from jax.experimental import pallas as pl
from jax.experimental.pallas import tpu as pltpu
from jax.experimental.pallas import tpu_sc as plsc

K = 50
N_NEG = 16384
N_POS = 4096
NC = 2
NS = 16
NW = NC * NS
CHUNK = N_NEG // NW
SLOTS = 64
LANES = 16


def _sc_topk_body(neg_hbm, vals_hbm, wts_hbm, chunk_v, vals_v, wts_v):
    wid = lax.axis_index("s") * NC + lax.axis_index("c")
    base = wid * CHUNK
    pltpu.sync_copy(neg_hbm.at[pl.ds(base, CHUNK)], chunk_v)

    neg_inf = jnp.float32(-jnp.inf)
    kf = jnp.float32(K)
    for i in range(SLOTS // LANES):
        vals_v[pl.ds(i * LANES, LANES)] = jnp.full((LANES,), neg_inf, jnp.float32)
        wts_v[pl.ds(i * LANES, LANES)] = jnp.zeros((LANES,), jnp.float32)

    lane0 = lax.iota(jnp.int32, LANES) == 0
    ones = jnp.ones((LANES,), jnp.float32)
    zeros = jnp.zeros((LANES,), jnp.float32)

    def it(k, carry):
        bound, cum = carry
        mv = jnp.full((LANES,), neg_inf, jnp.float32)
        cv = zeros
        for i in range(CHUNK // LANES):
            x = chunk_v[pl.ds(i * LANES, LANES)]
            mv = jnp.maximum(mv, jnp.where(x < bound, x, neg_inf))
            cv = cv + jnp.where(x == bound, ones, zeros)
        m = jnp.max(mv)
        c = jnp.sum(cv)
        active = jnp.logical_and(k >= 1, cum < kf)
        take = jnp.where(active, jnp.minimum(c, kf - cum), jnp.float32(0.0))
        vstore = jnp.where(active, bound, neg_inf)
        slot = jnp.full((LANES,), jnp.maximum(k - 1, 0), jnp.int32)
        plsc.store_scatter(vals_v, [slot], jnp.full((LANES,), vstore, jnp.float32), mask=lane0)
        plsc.store_scatter(wts_v, [slot], jnp.full((LANES,), take, jnp.float32), mask=lane0)
        new_bound = jnp.where(cum < kf, m, bound)
        return new_bound, cum + take

    lax.fori_loop(0, K + 1, it, (jnp.float32(jnp.inf), jnp.float32(0.0)))

    pltpu.sync_copy(vals_v, vals_hbm.at[pl.ds(wid * SLOTS, SLOTS)])
    pltpu.sync_copy(wts_v, wts_hbm.at[pl.ds(wid * SLOTS, SLOTS)])


@functools.cache
def _sc_topk():
    return pl.kernel(
        _sc_topk_body,
        mesh=plsc.VectorSubcoreMesh(
            core_axis_name="c", subcore_axis_name="s",
            num_cores=NC, num_subcores=NS),
        out_type=(
            jax.ShapeDtypeStruct((NW * SLOTS,), jnp.float32),
            jax.ShapeDtypeStruct((NW * SLOTS,), jnp.float32),
        ),
        scratch_types=(
            pltpu.VMEM((CHUNK,), jnp.float32),
            pltpu.VMEM((SLOTS,), jnp.float32),
            pltpu.VMEM((SLOTS,), jnp.float32),
        ),
        compiler_params=pltpu.CompilerParams(needs_layout_passes=False),
    )


def _tc_reduce_body(vals_ref, wts_ref, pos_ref, out_ref):
    vals = vals_ref[...]
    wts = wts_ref[...]
    pos = pos_ref[...]
    kf = jnp.float32(K)
    neg_inf = jnp.float32(-jnp.inf)

    def it(_, carry):
        bound, cum, acc = carry
        m = jnp.max(jnp.where(vals < bound, vals, neg_inf))
        c = jnp.sum(jnp.where(vals == m, wts, 0.0))
        active = cum < kf
        take = jnp.where(active, jnp.minimum(c, kf - cum), jnp.float32(0.0))
        x = pos - m
        bce = -jnp.log(jnp.clip(jax.nn.sigmoid(x), 1e-6, 1.0 - 1e-6))
        acc = acc + take * jnp.sum(bce)
        return jnp.where(active, m, bound), cum + take, acc

    _, _, acc = lax.fori_loop(
        0, K, it, (jnp.float32(jnp.inf), jnp.float32(0.0), jnp.float32(0.0)))
    out_ref[0, 0] = acc / jnp.float32(N_POS * N_NEG)


def kernel(score_neg, score_pos):
    cand_vals, cand_wts = _sc_topk()(score_neg)
    out = pl.pallas_call(
        _tc_reduce_body,
        out_shape=jax.ShapeDtypeStruct((1, 1), jnp.float32),
        out_specs=pl.BlockSpec(memory_space=pltpu.SMEM),
    )(
        cand_vals.reshape(16, 128),
        cand_wts.reshape(16, 128),
        score_pos.reshape(32, 128),
    )
    return out[0, 0]


# speedup vs baseline: 444.3490x; 1.1744x over previous
import functools

import jax
import jax.numpy as jnp
from jax import lax
from jax.experimental import pallas as pl
from jax.experimental.pallas import tpu as pltpu
from jax.experimental.pallas import tpu_sc as plsc

K = 50
N_NEG = 16384
N_POS = 4096
NC = 2
NS = 16
NW = NC * NS
CHUNK = N_NEG // NW
SLOTS = 64
LANES = 16


def _sc_topk_body(neg_hbm, vals_hbm, wts_hbm, chunk_v, vals_v, wts_v):
    wid = lax.axis_index("s") * NC + lax.axis_index("c")
    base = wid * CHUNK
    pltpu.sync_copy(neg_hbm.at[pl.ds(base, CHUNK)], chunk_v)

    neg_inf = jnp.float32(-jnp.inf)
    kf = jnp.float32(K)
    for i in range(SLOTS // LANES):
        vals_v[pl.ds(i * LANES, LANES)] = jnp.full((LANES,), neg_inf, jnp.float32)
        wts_v[pl.ds(i * LANES, LANES)] = jnp.zeros((LANES,), jnp.float32)

    lane0 = lax.iota(jnp.int32, LANES) == 0
    ones = jnp.ones((LANES,), jnp.float32)
    zeros = jnp.zeros((LANES,), jnp.float32)

    def it(k, carry):
        bound, cum = carry
        mv = jnp.full((LANES,), neg_inf, jnp.float32)
        cv = zeros
        for i in range(CHUNK // LANES):
            x = chunk_v[pl.ds(i * LANES, LANES)]
            mv = jnp.maximum(mv, jnp.where(x < bound, x, neg_inf))
            cv = cv + jnp.where(x == bound, ones, zeros)
        m = jnp.max(mv)
        c = jnp.sum(cv)
        active = jnp.logical_and(k >= 1, cum < kf)
        take = jnp.where(active, jnp.minimum(c, kf - cum), jnp.float32(0.0))
        vstore = jnp.where(active, bound, neg_inf)
        slot = jnp.full((LANES,), jnp.maximum(k - 1, 0), jnp.int32)
        plsc.store_scatter(vals_v, [slot], jnp.full((LANES,), vstore, jnp.float32), mask=lane0)
        plsc.store_scatter(wts_v, [slot], jnp.full((LANES,), take, jnp.float32), mask=lane0)
        new_bound = jnp.where(cum < kf, m, bound)
        return new_bound, cum + take

    lax.fori_loop(0, K + 1, it, (jnp.float32(jnp.inf), jnp.float32(0.0)))

    pltpu.sync_copy(vals_v, vals_hbm.at[pl.ds(wid * SLOTS, SLOTS)])
    pltpu.sync_copy(wts_v, wts_hbm.at[pl.ds(wid * SLOTS, SLOTS)])


@functools.cache
def _sc_topk():
    return pl.kernel(
        _sc_topk_body,
        mesh=plsc.VectorSubcoreMesh(
            core_axis_name="c", subcore_axis_name="s",
            num_cores=NC, num_subcores=NS),
        out_type=(
            jax.ShapeDtypeStruct((NW * SLOTS,), jnp.float32),
            jax.ShapeDtypeStruct((NW * SLOTS,), jnp.float32),
        ),
        scratch_types=(
            pltpu.VMEM((CHUNK,), jnp.float32),
            pltpu.VMEM((SLOTS,), jnp.float32),
            pltpu.VMEM((SLOTS,), jnp.float32),
        ),
        compiler_params=pltpu.CompilerParams(needs_layout_passes=False),
    )


def _tc_reduce_body(vals_ref, wts_ref, pos_ref, out_ref):
    vals = vals_ref[...]
    wts = wts_ref[...]
    pos = pos_ref[...]
    kf = jnp.float32(K)
    neg_inf = jnp.float32(-jnp.inf)

    rem = vals
    cum = jnp.float32(0.0)
    acc = jnp.float32(0.0)
    for _ in range(K):
        m = jnp.max(rem)
        c = jnp.sum(jnp.where(rem == m, wts, 0.0))
        rem = jnp.where(rem == m, neg_inf, rem)
        take = jnp.minimum(c, jnp.maximum(kf - cum, 0.0))
        cum = cum + c
        x = pos - m
        bce = -jnp.log(jnp.clip(jax.nn.sigmoid(x), 1e-6, 1.0 - 1e-6))
        acc = acc + take * jnp.sum(bce)
    out_ref[0, 0] = acc / jnp.float32(N_POS * N_NEG)


def kernel(score_neg, score_pos):
    cand_vals, cand_wts = _sc_topk()(score_neg)
    out = pl.pallas_call(
        _tc_reduce_body,
        out_shape=jax.ShapeDtypeStruct((1, 1), jnp.float32),
        out_specs=pl.BlockSpec(memory_space=pltpu.SMEM),
    )(
        cand_vals.reshape(16, 128),
        cand_wts.reshape(16, 128),
        score_pos.reshape(32, 128),
    )
    return out[0, 0]


# speedup vs baseline: 445.0329x; 1.0015x over previous
import functools

import jax
import jax.numpy as jnp
from jax import lax
from jax.experimental import pallas as pl
from jax.experimental.pallas import tpu as pltpu
from jax.experimental.pallas import tpu_sc as plsc

K = 50
N_NEG = 16384
N_POS = 4096
NC = 2
NS = 16
NW = NC * NS
CHUNK = N_NEG // NW
SLOTS = 64
LANES = 16


def _sc_topk_body(neg_hbm, vals_hbm, wts_hbm, chunk_v, vals_v, wts_v):
    wid = lax.axis_index("s") * NC + lax.axis_index("c")
    base = wid * CHUNK
    pltpu.sync_copy(neg_hbm.at[pl.ds(base, CHUNK)], chunk_v)

    neg_inf = jnp.float32(-jnp.inf)
    kf = jnp.float32(K)
    for i in range(SLOTS // LANES):
        vals_v[pl.ds(i * LANES, LANES)] = jnp.full((LANES,), neg_inf, jnp.float32)
        wts_v[pl.ds(i * LANES, LANES)] = jnp.zeros((LANES,), jnp.float32)

    lane0 = lax.iota(jnp.int32, LANES) == 0
    ones = jnp.ones((LANES,), jnp.float32)
    zeros = jnp.zeros((LANES,), jnp.float32)

    def it(k, carry):
        bound, cum = carry
        mv = jnp.full((LANES,), neg_inf, jnp.float32)
        cv = zeros
        for i in range(CHUNK // LANES):
            x = chunk_v[pl.ds(i * LANES, LANES)]
            mv = jnp.maximum(mv, jnp.where(x < bound, x, neg_inf))
            cv = cv + jnp.where(x == bound, ones, zeros)
        m = jnp.max(mv)
        c = jnp.sum(cv)
        active = jnp.logical_and(k >= 1, cum < kf)
        take = jnp.where(active, jnp.minimum(c, kf - cum), jnp.float32(0.0))
        vstore = jnp.where(active, bound, neg_inf)
        slot = jnp.full((LANES,), jnp.maximum(k - 1, 0), jnp.int32)
        plsc.store_scatter(vals_v, [slot], jnp.full((LANES,), vstore, jnp.float32), mask=lane0)
        plsc.store_scatter(wts_v, [slot], jnp.full((LANES,), take, jnp.float32), mask=lane0)
        new_bound = jnp.where(cum < kf, m, bound)
        return new_bound, cum + take

    lax.fori_loop(0, K + 1, it, (jnp.float32(jnp.inf), jnp.float32(0.0)))

    pltpu.sync_copy(vals_v, vals_hbm.at[pl.ds(wid * SLOTS, SLOTS)])
    pltpu.sync_copy(wts_v, wts_hbm.at[pl.ds(wid * SLOTS, SLOTS)])


@functools.cache
def _sc_topk():
    return pl.kernel(
        _sc_topk_body,
        mesh=plsc.VectorSubcoreMesh(
            core_axis_name="c", subcore_axis_name="s",
            num_cores=NC, num_subcores=NS),
        out_type=(
            jax.ShapeDtypeStruct((NW * SLOTS,), jnp.float32),
            jax.ShapeDtypeStruct((NW * SLOTS,), jnp.float32),
        ),
        scratch_types=(
            pltpu.VMEM((CHUNK,), jnp.float32),
            pltpu.VMEM((SLOTS,), jnp.float32),
            pltpu.VMEM((SLOTS,), jnp.float32),
        ),
        compiler_params=pltpu.CompilerParams(
            needs_layout_passes=False, skip_device_barrier=True),
    )


def _tc_reduce_body(vals_ref, wts_ref, pos_ref, out_ref):
    vals = vals_ref[...]
    wts = wts_ref[...]
    pos = pos_ref[...]
    kf = jnp.float32(K)
    neg_inf = jnp.float32(-jnp.inf)

    rem = vals
    cum = jnp.float32(0.0)
    acc = jnp.float32(0.0)
    for _ in range(K):
        m = jnp.max(rem)
        c = jnp.sum(jnp.where(rem == m, wts, 0.0))
        rem = jnp.where(rem == m, neg_inf, rem)
        take = jnp.minimum(c, jnp.maximum(kf - cum, 0.0))
        cum = cum + c
        x = pos - m
        bce = -jnp.log(jnp.clip(jax.nn.sigmoid(x), 1e-6, 1.0 - 1e-6))
        acc = acc + take * jnp.sum(bce)
    out_ref[0, 0] = acc / jnp.float32(N_POS * N_NEG)


def kernel(score_neg, score_pos):
    cand_vals, cand_wts = _sc_topk()(score_neg)
    out = pl.pallas_call(
        _tc_reduce_body,
        out_shape=jax.ShapeDtypeStruct((1, 1), jnp.float32),
        out_specs=pl.BlockSpec(memory_space=pltpu.SMEM),
    )(
        cand_vals.reshape(16, 128),
        cand_wts.reshape(16, 128),
        score_pos.reshape(32, 128),
    )
    return out[0, 0]
